# baseline (device time: 26471 ns/iter reference)
import jax
import jax.numpy as jnp
from jax import lax
from jax.experimental import pallas as pl
from jax.experimental.pallas import tpu as pltpu

N_DEV = 32
N_TOK = 512
D_IN = 256
D_OUT = 512
N_EXP = 64
CAP = 6
TOK_PER_DEV = N_TOK // N_DEV
EXP_PER_DEV = N_EXP // N_DEV


def kernel(x, router_W, route_idx, expert_W):
    del router_W

    def body(x_ref, ridx_ref, w_ref, out_ref,
             yb_ref, recv_ref, send_sems, recv_sems):
        me = lax.axis_index("i")

        bar = pltpu.get_barrier_semaphore()
        for off in range(1, N_DEV):
            t = lax.rem(me + off, N_DEV)
            pl.semaphore_signal(bar, inc=1, device_id=(t,),
                                device_id_type=pl.DeviceIdType.MESH)
        pl.semaphore_wait(bar, N_DEV - 1)

        ridx = ridx_ref[:, :]
        eids = lax.broadcasted_iota(jnp.int32, (N_TOK, N_EXP), 1)
        onehot = (ridx == eids).astype(jnp.float32)
        row = lax.broadcasted_iota(jnp.int32, (N_TOK, N_TOK), 0)
        col = lax.broadcasted_iota(jnp.int32, (N_TOK, N_TOK), 1)
        lower = (col <= row).astype(jnp.float32)
        incl = jnp.dot(lower, onehot, preferred_element_type=jnp.float32)
        rank = jnp.sum(incl * onehot, axis=1, keepdims=True)
        keep = rank <= (CAP + 0.5)

        x_val = x_ref[:, :]
        y = jnp.zeros((N_TOK, D_OUT), jnp.float32)
        for k in range(EXP_PER_DEV):
            e = me * EXP_PER_DEV + k
            m = jnp.logical_and(ridx == e, keep).astype(jnp.float32)
            xm = (x_val * m).astype(jnp.bfloat16)
            y = y + jnp.dot(xm, w_ref[k].astype(jnp.bfloat16),
                            preferred_element_type=jnp.float32)

        perm = (col == lax.rem(row + me * TOK_PER_DEV, N_TOK)).astype(
            jnp.float32)
        z = jnp.dot(perm, y, preferred_element_type=jnp.float32)
        yb_ref[...] = z.reshape(N_DEV, TOK_PER_DEV, D_OUT)

        rdmas = []
        for off in range(1, N_DEV):
            t = lax.rem(me + off, N_DEV)
            rdma = pltpu.make_async_remote_copy(
                src_ref=yb_ref.at[off],
                dst_ref=recv_ref.at[off],
                send_sem=send_sems.at[off],
                recv_sem=recv_sems.at[off],
                device_id=(t,),
                device_id_type=pl.DeviceIdType.MESH,
            )
            rdma.start()
            rdmas.append(rdma)

        acc = yb_ref[0]
        for off in range(1, N_DEV):
            rdmas[off - 1].wait_recv()
            acc = acc + recv_ref[off]
        out_ref[...] = acc

        for rdma in rdmas:
            rdma.wait_send()

    return pl.pallas_call(
        body,
        out_shape=jax.ShapeDtypeStruct((TOK_PER_DEV, D_OUT), jnp.float32),
        in_specs=[pl.BlockSpec(memory_space=pltpu.VMEM)] * 3,
        out_specs=pl.BlockSpec(memory_space=pltpu.VMEM),
        scratch_shapes=[
            pltpu.VMEM((N_DEV, TOK_PER_DEV, D_OUT), jnp.float32),
            pltpu.VMEM((N_DEV, TOK_PER_DEV, D_OUT), jnp.float32),
            pltpu.SemaphoreType.DMA((N_DEV,)),
            pltpu.SemaphoreType.DMA((N_DEV,)),
        ],
        compiler_params=pltpu.CompilerParams(collective_id=0),
    )(x, route_idx, expert_W)


# device time: 21239 ns/iter; 1.2463x vs baseline; 1.2463x over previous
import jax
import jax.numpy as jnp
from jax import lax
from jax.experimental import pallas as pl
from jax.experimental.pallas import tpu as pltpu

N_DEV = 32
N_TOK = 512
D_IN = 256
D_OUT = 512
N_EXP = 64
CAP = 6
TOK_PER_DEV = N_TOK // N_DEV
EXP_PER_DEV = N_EXP // N_DEV


def kernel(x, router_W, route_idx, expert_W):
    del router_W

    def body(x_ref, ridx_ref, w_ref, out_ref,
             yb_ref, recv_ref, send_sems, recv_sems):
        me = lax.axis_index("i")

        bar = pltpu.get_barrier_semaphore()
        for off in range(1, N_DEV):
            t = lax.rem(me + off, N_DEV)
            pl.semaphore_signal(bar, inc=1, device_id=(t,),
                                device_id_type=pl.DeviceIdType.MESH)
        pl.semaphore_wait(bar, N_DEV - 1)

        ridx = ridx_ref[:, :]
        eids = lax.broadcasted_iota(jnp.int32, (N_TOK, N_EXP), 1)
        onehot = (ridx == eids).astype(jnp.bfloat16)
        row = lax.broadcasted_iota(jnp.int32, (N_TOK, N_TOK), 0)
        col = lax.broadcasted_iota(jnp.int32, (N_TOK, N_TOK), 1)
        lower = (col <= row).astype(jnp.bfloat16)
        incl = jnp.dot(lower, onehot, preferred_element_type=jnp.float32)
        rank = jnp.sum(incl * onehot.astype(jnp.float32), axis=1,
                       keepdims=True)
        keep = rank <= (CAP + 0.5)

        x_val = x_ref[:, :]
        y = jnp.zeros((N_TOK, D_OUT), jnp.float32)
        for k in range(EXP_PER_DEV):
            e = me * EXP_PER_DEV + k
            m = jnp.logical_and(ridx == e, keep).astype(jnp.float32)
            xm = (x_val * m).astype(jnp.bfloat16)
            y = y + jnp.dot(xm, w_ref[k].astype(jnp.bfloat16),
                            preferred_element_type=jnp.float32)

        perm = (col == lax.rem(row + me * TOK_PER_DEV, N_TOK)).astype(
            jnp.bfloat16)
        z = jnp.dot(perm, y.astype(jnp.bfloat16),
                    preferred_element_type=jnp.float32)
        yb_ref[...] = z.astype(jnp.bfloat16).reshape(
            N_DEV, TOK_PER_DEV, D_OUT)

        rdmas = []
        for off in range(1, N_DEV):
            t = lax.rem(me + off, N_DEV)
            rdma = pltpu.make_async_remote_copy(
                src_ref=yb_ref.at[off],
                dst_ref=recv_ref.at[off],
                send_sem=send_sems.at[off],
                recv_sem=recv_sems.at[off],
                device_id=(t,),
                device_id_type=pl.DeviceIdType.MESH,
            )
            rdma.start()
            rdmas.append(rdma)

        acc = yb_ref[0].astype(jnp.float32)
        for off in range(1, N_DEV):
            rdmas[off - 1].wait_recv()
            acc = acc + recv_ref[off].astype(jnp.float32)
        out_ref[...] = acc

        for rdma in rdmas:
            rdma.wait_send()

    return pl.pallas_call(
        body,
        out_shape=jax.ShapeDtypeStruct((TOK_PER_DEV, D_OUT), jnp.float32),
        in_specs=[pl.BlockSpec(memory_space=pltpu.VMEM)] * 3,
        out_specs=pl.BlockSpec(memory_space=pltpu.VMEM),
        scratch_shapes=[
            pltpu.VMEM((N_DEV, TOK_PER_DEV, D_OUT), jnp.bfloat16),
            pltpu.VMEM((N_DEV, TOK_PER_DEV, D_OUT), jnp.bfloat16),
            pltpu.SemaphoreType.DMA((N_DEV,)),
            pltpu.SemaphoreType.DMA((N_DEV,)),
        ],
        compiler_params=pltpu.CompilerParams(collective_id=0),
    )(x, route_idx, expert_W)


# device time: 19688 ns/iter; 1.3445x vs baseline; 1.0788x over previous
import os

import jax
import jax.numpy as jnp
from jax import lax
from jax.experimental import pallas as pl
from jax.experimental.pallas import tpu as pltpu

N_DEV = 32
N_TOK = 512
D_IN = 256
D_OUT = 512
N_EXP = 64
CAP = 6
TOK_PER_DEV = N_TOK // N_DEV
EXP_PER_DEV = N_EXP // N_DEV
N_CHUNK = 4
DEV_PER_CHUNK = N_DEV // N_CHUNK
ROWS_PER_CHUNK = N_TOK // N_CHUNK

_KVAR = os.environ.get("KVAR", "full")


def kernel(x, router_W, route_idx, expert_W):
    del router_W

    def body(x_ref, ridx_ref, w_ref, out_ref,
             xm_ref, yb_ref, recv_ref, alive_sems, send_sems, recv_sems):
        me = lax.axis_index("i")
        comm = _KVAR != "nocomm"

        if comm:
            bar = pltpu.get_barrier_semaphore()
            pl.semaphore_signal(bar, inc=1)
            pl.semaphore_wait(bar, 1)
            for off in range(1, N_DEV):
                t = lax.rem(me + off, N_DEV)
                pl.semaphore_signal(
                    alive_sems.at[N_DEV - off], inc=1, device_id=(t,),
                    device_id_type=pl.DeviceIdType.MESH)

        ridx = ridx_ref[:, :]
        eids = lax.broadcasted_iota(jnp.int32, (N_TOK, N_EXP), 1)
        onehot = (ridx == eids).astype(jnp.float32)
        incl = onehot
        sh = 1
        while sh < N_TOK:
            incl = incl + jnp.concatenate(
                [jnp.zeros((sh, N_EXP), jnp.float32), incl[:-sh]], axis=0)
            sh *= 2
        rank = jnp.sum(incl * onehot, axis=1, keepdims=True)
        keep = rank <= (CAP + 0.5)

        x_val = x_ref[:, :]
        cols = []
        for k in range(EXP_PER_DEV):
            e = me * EXP_PER_DEV + k
            m = jnp.logical_and(ridx == e, keep).astype(jnp.float32)
            cols.append((x_val * m).astype(jnp.bfloat16))
        xmm = jnp.concatenate(cols, axis=1)
        xm_ref[...] = jnp.concatenate([xmm, xmm], axis=0)

        wcat = jnp.concatenate(
            [w_ref[k].astype(jnp.bfloat16) for k in range(EXP_PER_DEV)],
            axis=0)

        rdmas = []
        if _KVAR == "stale":
            for off in range(1, N_DEV):
                t = lax.rem(me + off, N_DEV)
                pl.semaphore_wait(alive_sems.at[off], 1)
                rdma = pltpu.make_async_remote_copy(
                    src_ref=yb_ref.at[off], dst_ref=recv_ref.at[off],
                    send_sem=send_sems.at[off], recv_sem=recv_sems.at[off],
                    device_id=(t,), device_id_type=pl.DeviceIdType.MESH)
                rdma.start()
                rdmas.append(rdma)
        for c in range(N_CHUNK):
            start = me * TOK_PER_DEV + c * ROWS_PER_CHUNK
            xc = xm_ref[pl.ds(start, ROWS_PER_CHUNK), :]
            yc = jnp.dot(xc, wcat, preferred_element_type=jnp.float32)
            yb_ref[c * DEV_PER_CHUNK:(c + 1) * DEV_PER_CHUNK] = (
                yc.astype(jnp.bfloat16).reshape(
                    DEV_PER_CHUNK, TOK_PER_DEV, D_OUT))
            if not comm or _KVAR == "stale":
                continue
            for off in range(c * DEV_PER_CHUNK, (c + 1) * DEV_PER_CHUNK):
                if off == 0:
                    continue
                t = lax.rem(me + off, N_DEV)
                pl.semaphore_wait(alive_sems.at[off], 1)
                rdma = pltpu.make_async_remote_copy(
                    src_ref=yb_ref.at[off],
                    dst_ref=recv_ref.at[off],
                    send_sem=send_sems.at[off],
                    recv_sem=recv_sems.at[off],
                    device_id=(t,),
                    device_id_type=pl.DeviceIdType.MESH,
                )
                rdma.start()
                rdmas.append(rdma)

        acc = yb_ref[0].astype(jnp.float32)
        if comm:
            for off in range(1, N_DEV):
                rdmas[off - 1].wait_recv()
                acc = acc + recv_ref[off].astype(jnp.float32)
        out_ref[...] = acc

        for rdma in rdmas:
            rdma.wait_send()

    return pl.pallas_call(
        body,
        out_shape=jax.ShapeDtypeStruct((TOK_PER_DEV, D_OUT), jnp.float32),
        in_specs=[pl.BlockSpec(memory_space=pltpu.VMEM)] * 3,
        out_specs=pl.BlockSpec(memory_space=pltpu.VMEM),
        scratch_shapes=[
            pltpu.VMEM((2 * N_TOK, 2 * D_IN), jnp.bfloat16),
            pltpu.VMEM((N_DEV, TOK_PER_DEV, D_OUT), jnp.bfloat16),
            pltpu.VMEM((N_DEV, TOK_PER_DEV, D_OUT), jnp.bfloat16),
            pltpu.SemaphoreType.REGULAR((N_DEV,)),
            pltpu.SemaphoreType.DMA((N_DEV,)),
            pltpu.SemaphoreType.DMA((N_DEV,)),
        ],
        compiler_params=(pltpu.CompilerParams() if _KVAR == "nocomm"
                         else pltpu.CompilerParams(collective_id=0)),
    )(x, route_idx, expert_W)
